# baseline (speedup 1.0000x reference)
import functools

import jax
import jax.numpy as jnp
from jax import lax
from jax.experimental import pallas as pl
from jax.experimental.pallas import tpu as pltpu
from jax.experimental.pallas import tpu_sc as plsc

N_SAE = 8
D_DATA = 256
D_DICT = 1024
TOKENS = 2048
TOP_K = 2
EB = 256
NBLK = 23
NPAD = NBLK * EB
NPAIR = TOKENS * TOP_K

NC = 2
NS = 16
NW = NC * NS
PAIRS_PER_W = NPAIR // NW
TOK_PER_W = TOKENS // NW


def _wid():
    return lax.axis_index("s") * NC + lax.axis_index("c")


@functools.cache
def _sc_kernels():
    mesh = plsc.VectorSubcoreMesh(
        core_axis_name="c", subcore_axis_name="s",
        num_cores=NC, num_subcores=NS)

    @functools.partial(
        pl.kernel,
        out_type=(
            jax.ShapeDtypeStruct((NPAD, D_DATA), jnp.float32),
            jax.ShapeDtypeStruct((NPAD, 128), jnp.float32),
        ),
        mesh=mesh,
        scratch_types=[
            pltpu.VMEM((PAIRS_PER_W,), jnp.int32),
            pltpu.VMEM((PAIRS_PER_W,), jnp.int32),
            pltpu.VMEM((PAIRS_PER_W, D_DATA), jnp.float32),
            pltpu.VMEM((PAIRS_PER_W, 128), jnp.float32),
            pltpu.SemaphoreType.DMA,
        ],
    )
    def _dispatch(x_hbm, wr_hbm, tok_hbm, dst_hbm, xs_hbm, wp_hbm,
                  tv, dv, rows, wv, sem):
        base = _wid() * PAIRS_PER_W
        pltpu.sync_copy(tok_hbm.at[pl.ds(base, PAIRS_PER_W)], tv)
        pltpu.sync_copy(dst_hbm.at[pl.ds(base, PAIRS_PER_W)], dv)
        pltpu.sync_copy(wr_hbm.at[pl.ds(base, PAIRS_PER_W)], wv)
        gather = pltpu.async_copy(x_hbm.at[tv], rows, sem)
        gather.wait()
        sc1 = pltpu.async_copy(rows, xs_hbm.at[dv], sem)
        sc2 = pltpu.async_copy(wv, wp_hbm.at[dv], sem)
        sc1.wait()
        sc2.wait()

    @functools.partial(
        pl.kernel,
        out_type=jax.ShapeDtypeStruct((TOKENS, D_DATA), jnp.float32),
        mesh=mesh,
        scratch_types=[
            pltpu.VMEM((TOK_PER_W,), jnp.int32),
            pltpu.VMEM((TOK_PER_W,), jnp.int32),
            pltpu.VMEM((TOK_PER_W, D_DATA), jnp.float32),
            pltpu.VMEM((TOK_PER_W, D_DATA), jnp.float32),
            pltpu.SemaphoreType.DMA,
        ],
    )
    def _combine(y_hbm, p0_hbm, p1_hbm, out_hbm, p0v, p1v, buf0, buf1, sem):
        tbase = _wid() * TOK_PER_W
        pltpu.sync_copy(p0_hbm.at[pl.ds(tbase, TOK_PER_W)], p0v)
        pltpu.sync_copy(p1_hbm.at[pl.ds(tbase, TOK_PER_W)], p1v)
        cp0 = pltpu.async_copy(y_hbm.at[p0v], buf0, sem)
        cp1 = pltpu.async_copy(y_hbm.at[p1v], buf1, sem)
        cp0.wait()
        cp1.wait()

        def body(t, carry):
            for c in range(D_DATA // 16):
                sl = pl.ds(c * 16, 16)
                buf0[t, sl] = buf0[t, sl] + buf1[t, sl]
            return carry

        lax.fori_loop(0, TOK_PER_W, body, 0)
        pltpu.sync_copy(buf0, out_hbm.at[pl.ds(tbase, TOK_PER_W)])

    return _dispatch, _combine


def _mm_body(eob_ref, nused_ref, xs_ref, wp_ref, we_ref, be_ref, wd_ref,
             bd_ref, y_ref):
    del eob_ref

    @pl.when(pl.program_id(0) < nused_ref[0])
    def _():
        acts = jnp.maximum(
            jnp.dot(xs_ref[...].astype(jnp.bfloat16), we_ref[0],
                    preferred_element_type=jnp.float32)
            + be_ref[0],
            0.0,
        )
        dec = jnp.dot(acts.astype(jnp.bfloat16), wd_ref[0],
                      preferred_element_type=jnp.float32)
        y_ref[...] = dec * wp_ref[:, :1] + bd_ref[0]


def _grouped_mm(x_sorted, w_padded, eob, nused, W_enc, b_enc, W_dec, b_dec):
    return pl.pallas_call(
        _mm_body,
        grid_spec=pltpu.PrefetchScalarGridSpec(
            num_scalar_prefetch=2,
            grid=(NBLK,),
            in_specs=[
                pl.BlockSpec((EB, D_DATA), lambda i, eob, nu: (i, 0)),
                pl.BlockSpec((EB, 128), lambda i, eob, nu: (i, 0)),
                pl.BlockSpec((1, D_DATA, D_DICT),
                             lambda i, eob, nu: (eob[i], 0, 0)),
                pl.BlockSpec((1, 1, D_DICT), lambda i, eob, nu: (eob[i], 0, 0)),
                pl.BlockSpec((1, D_DICT, D_DATA),
                             lambda i, eob, nu: (eob[i], 0, 0)),
                pl.BlockSpec((1, 1, D_DATA), lambda i, eob, nu: (eob[i], 0, 0)),
            ],
            out_specs=pl.BlockSpec((EB, D_DATA), lambda i, eob, nu: (i, 0)),
        ),
        out_shape=jax.ShapeDtypeStruct((NPAD, D_DATA), jnp.float32),
        compiler_params=pltpu.CompilerParams(
            dimension_semantics=("arbitrary",),
        ),
    )(eob, nused, x_sorted, w_padded, W_enc.astype(jnp.bfloat16),
      b_enc.reshape(N_SAE, 1, D_DICT), W_dec.astype(jnp.bfloat16),
      b_dec.reshape(N_SAE, 1, D_DATA))


def _routing(gate):
    iota8 = jnp.arange(N_SAE, dtype=jnp.int32)
    e0 = jnp.argmax(gate, axis=1).astype(jnp.int32)
    oh0 = (e0[:, None] == iota8[None, :]).astype(gate.dtype)
    g0 = jnp.sum(gate * oh0, axis=1)
    gate1 = gate * (1.0 - oh0)
    e1 = jnp.argmax(gate1, axis=1).astype(jnp.int32)
    oh1 = (e1[:, None] == iota8[None, :]).astype(gate.dtype)
    g1 = jnp.sum(gate1 * oh1, axis=1)

    e_flat = jnp.stack([e0, e1], axis=1).reshape(-1)
    oh = (e_flat[:, None] == iota8[None, :]).astype(jnp.int32)
    cum = jnp.cumsum(oh, axis=0)
    counts = cum[-1]
    rank = jnp.sum(cum * oh, axis=1) - 1
    nblk = (counts + EB - 1) // EB
    cumblk = jnp.cumsum(nblk)
    row_start = (cumblk - nblk) * EB
    dst = (jnp.sum(row_start[None, :] * oh, axis=1)
           + rank).astype(jnp.int32)
    blk_iota = jnp.arange(NBLK, dtype=jnp.int32)
    eob = jnp.sum((cumblk[None, :] <= blk_iota[:, None]).astype(jnp.int32),
                  axis=1)
    eob = jnp.minimum(eob, N_SAE - 1).astype(jnp.int32)
    nused = cumblk[-1:].astype(jnp.int32)
    tok_flat = (jnp.arange(NPAIR, dtype=jnp.int32) // TOP_K)
    w_flat = jnp.stack([g0, g1], axis=1).reshape(-1)
    return tok_flat, dst, w_flat, eob, nused


def kernel(x, gate, W_enc, b_enc, W_dec, b_dec):
    dispatch, combine = _sc_kernels()
    tok_flat, dst, w_flat, eob, nused = _routing(gate)
    w_rows = jnp.broadcast_to(w_flat[:, None], (NPAIR, 128))
    x_sorted, w_padded = dispatch(x, w_rows, tok_flat, dst)
    y = _grouped_mm(x_sorted, w_padded, eob, nused, W_enc, b_enc, W_dec, b_dec)
    pos = dst.reshape(TOKENS, TOP_K)
    out = combine(y, pos[:, 0], pos[:, 1])
    return out


# speedup vs baseline: 2.5457x; 2.5457x over previous
import functools

import jax
import jax.numpy as jnp
from jax import lax
from jax.experimental import pallas as pl
from jax.experimental.pallas import tpu as pltpu
from jax.experimental.pallas import tpu_sc as plsc

N_SAE = 8
D_DATA = 256
D_DICT = 1024
TOKENS = 2048
TB = 256


def _dense_body(x_ref, g_ref, we_ref, be_ref, wd_ref, bd_ref, o_ref):
    x = x_ref[...].astype(jnp.bfloat16)
    g = g_ref[...]
    acc = jnp.zeros((TB, D_DATA), jnp.float32)
    for s in range(N_SAE):
        acts = jnp.maximum(
            jnp.dot(x, we_ref[s].astype(jnp.bfloat16),
                    preferred_element_type=jnp.float32)
            + be_ref[s][None, :],
            0.0,
        )
        gs = g[:, s:s + 1]
        dec = jnp.dot((acts * gs).astype(jnp.bfloat16),
                      wd_ref[s].astype(jnp.bfloat16),
                      preferred_element_type=jnp.float32)
        msk = (gs != 0.0).astype(jnp.float32)
        acc = acc + dec + msk * bd_ref[s][None, :]
    o_ref[...] = acc


def kernel(x, gate, W_enc, b_enc, W_dec, b_dec):
    grid = (TOKENS // TB,)
    out = pl.pallas_call(
        _dense_body,
        grid=grid,
        in_specs=[
            pl.BlockSpec((TB, D_DATA), lambda i: (i, 0)),
            pl.BlockSpec((TB, N_SAE), lambda i: (i, 0)),
            pl.BlockSpec((N_SAE, D_DATA, D_DICT), lambda i: (0, 0, 0)),
            pl.BlockSpec((N_SAE, D_DICT), lambda i: (0, 0)),
            pl.BlockSpec((N_SAE, D_DICT, D_DATA), lambda i: (0, 0, 0)),
            pl.BlockSpec((N_SAE, D_DATA), lambda i: (0, 0)),
        ],
        out_specs=pl.BlockSpec((TB, D_DATA), lambda i: (i, 0)),
        out_shape=jax.ShapeDtypeStruct((TOKENS, D_DATA), jnp.float32),
        compiler_params=pltpu.CompilerParams(
            dimension_semantics=("parallel",),
        ),
    )(x, gate, W_enc, b_enc, W_dec, b_dec)
    return out
